# baseline (speedup 1.0000x reference)
import jax
import jax.numpy as jnp
from jax.experimental import pallas as pl

_T = 0.07


def _norm_kernel(x_ref, o_ref):
    x = x_ref[...]
    o_ref[...] = x / jnp.sqrt(jnp.sum(x * x, axis=1, keepdims=True))


def kernel(points, point_indices, memory_bank):
    norm_points = pl.pallas_call(
        _norm_kernel,
        out_shape=jax.ShapeDtypeStruct(points.shape, points.dtype),
    )(points)
    similarities = norm_points @ memory_bank.T
    points_sim = jnp.exp(similarities / _T)
    positive_sim = points_sim[jnp.arange(points_sim.shape[0]), point_indices]
    hard_negatives_sim, _ = jax.lax.top_k(points_sim, 4096)
    potential_positive_sim = hard_negatives_sim[:, :100]
    total_positive_sim = positive_sim + potential_positive_sim.sum(axis=1)
    loss = -jnp.log(total_positive_sim / hard_negatives_sim.sum(axis=1) + 1e-07).mean()
    return (loss, jax.lax.stop_gradient(similarities))


# speedup vs baseline: 11.4453x; 11.4453x over previous
import jax
import jax.numpy as jnp
from jax.experimental import pallas as pl
from jax.experimental.pallas import tpu as pltpu

_T = 0.07
_K_HI = 100.0
_K_LO = 4096.0
_BISECT_ITERS = 27


def _k1_body(ncb, n_cols):
    def body(p_ref, bank_ref, idx_ref, sims_ref, m_ref, s_ref, pos_ref,
             accm, accs, accp):
        cb = pl.program_id(1)
        p = p_ref[...]
        npts = p / jnp.sqrt(jnp.sum(p * p, axis=1, keepdims=True))
        bank = bank_ref[...]
        s = jax.lax.dot_general(
            npts, bank, (((1,), (1,)), ((), ())),
            preferred_element_type=jnp.float32,
            precision=jax.lax.Precision.DEFAULT)
        sims_ref[...] = s

        bc = s.shape[1]
        cg = cb * bc + jax.lax.broadcasted_iota(jnp.int32, s.shape, 1)
        valid = cg < n_cols
        e = jnp.exp(s / _T)
        ez = jnp.where(valid, e, 0.0)
        sz = jnp.where(valid, s, -jnp.inf)

        @pl.when(cb == 0)
        def _init():
            accm[...] = jnp.full(accm.shape, -jnp.inf, jnp.float32)
            accs[...] = jnp.zeros(accs.shape, jnp.float32)
            accp[...] = jnp.zeros(accp.shape, jnp.float32)

        accm[...] = jnp.maximum(accm[...], jnp.max(sz, axis=1, keepdims=True))
        accs[...] = accs[...] + jnp.sum(ez, axis=1, keepdims=True)
        hit = (cg == idx_ref[...]) & valid
        accp[...] = accp[...] + jnp.sum(jnp.where(hit, e, 0.0), axis=1,
                                        keepdims=True)

        @pl.when(cb == ncb - 1)
        def _fin():
            m_ref[...] = accm[...]
            s_ref[...] = accs[...]
            pos_ref[...] = accp[...]

    return body


def _k2_body(nrb2, n_rows):
    def body(sims_ref, m_ref, pos_ref, out_ref, e_scr):
        i = pl.program_id(0)
        s = sims_ref[...]
        m = m_ref[...]
        pos = pos_ref[...]

        lo0 = m - 5.0
        hi0 = m + 0.001

        def bis(_, carry):
            loa, hia, lob, hib = carry
            mida = 0.5 * (loa + hia)
            midb = 0.5 * (lob + hib)
            cnta = jnp.sum((s >= mida).astype(jnp.float32), axis=1,
                           keepdims=True)
            cntb = jnp.sum((s >= midb).astype(jnp.float32), axis=1,
                           keepdims=True)
            preda = cnta >= _K_HI
            predb = cntb >= _K_LO
            return (jnp.where(preda, mida, loa), jnp.where(preda, hia, mida),
                    jnp.where(predb, midb, lob), jnp.where(predb, hib, midb))

        loa, hia, lob, hib = jax.lax.fori_loop(
            0, _BISECT_ITERS, bis, (lo0, hi0, lo0, hi0))

        e_scr[...] = jnp.exp(s / _T)
        e = e_scr[...]
        maska = s >= hia
        c_a = jnp.sum(maska.astype(jnp.float32), axis=1, keepdims=True)
        e_a = jnp.sum(jnp.where(maska, e, 0.0), axis=1, keepdims=True)
        s100 = e_a + (_K_HI - c_a) * jnp.exp(loa / _T)
        maskb = s >= hib
        c_b = jnp.sum(maskb.astype(jnp.float32), axis=1, keepdims=True)
        e_b = jnp.sum(jnp.where(maskb, e, 0.0), axis=1, keepdims=True)
        s4096 = e_b + (_K_LO - c_b) * jnp.exp(lob / _T)
        ratio = (pos + s100) / s4096
        term = jnp.log(ratio + 1e-7)

        @pl.when(i == 0)
        def _init():
            out_ref[...] = jnp.zeros((1, 1), jnp.float32)

        out_ref[...] = out_ref[...] + jnp.sum(term).reshape(1, 1)

        @pl.when(i == nrb2 - 1)
        def _fin():
            out_ref[...] = -out_ref[...] / n_rows

    return body


def kernel(points, point_indices, memory_bank):
    b, d = points.shape
    n = memory_bank.shape[0]
    r1 = min(512, b)
    bc = 512
    nrb = b // r1
    ncb = pl.cdiv(n, bc)
    idx2 = point_indices.reshape(b, 1)

    sims, m, stot, pos = pl.pallas_call(
        _k1_body(ncb, n),
        grid=(nrb, ncb),
        in_specs=[
            pl.BlockSpec((r1, d), lambda rb, cb: (rb, 0)),
            pl.BlockSpec((bc, d), lambda rb, cb: (cb, 0)),
            pl.BlockSpec((r1, 1), lambda rb, cb: (rb, 0)),
        ],
        out_specs=[
            pl.BlockSpec((r1, bc), lambda rb, cb: (rb, cb)),
            pl.BlockSpec((r1, 1), lambda rb, cb: (rb, 0)),
            pl.BlockSpec((r1, 1), lambda rb, cb: (rb, 0)),
            pl.BlockSpec((r1, 1), lambda rb, cb: (rb, 0)),
        ],
        out_shape=[
            jax.ShapeDtypeStruct((b, n), jnp.float32),
            jax.ShapeDtypeStruct((b, 1), jnp.float32),
            jax.ShapeDtypeStruct((b, 1), jnp.float32),
            jax.ShapeDtypeStruct((b, 1), jnp.float32),
        ],
        scratch_shapes=[
            pltpu.VMEM((r1, 1), jnp.float32),
            pltpu.VMEM((r1, 1), jnp.float32),
            pltpu.VMEM((r1, 1), jnp.float32),
        ],
    )(points, memory_bank, idx2)

    r2 = min(8, b)
    nrb2 = b // r2
    loss = pl.pallas_call(
        _k2_body(nrb2, b),
        grid=(nrb2,),
        in_specs=[
            pl.BlockSpec((r2, n), lambda i: (i, 0)),
            pl.BlockSpec((r2, 1), lambda i: (i, 0)),
            pl.BlockSpec((r2, 1), lambda i: (i, 0)),
        ],
        out_specs=pl.BlockSpec((1, 1), lambda i: (0, 0)),
        out_shape=jax.ShapeDtypeStruct((1, 1), jnp.float32),
        scratch_shapes=[pltpu.VMEM((r2, n), jnp.float32)],
    )(sims, m, pos)

    return (loss[0, 0], jax.lax.stop_gradient(sims))


# speedup vs baseline: 14.0943x; 1.2314x over previous
import jax
import jax.numpy as jnp
from jax.experimental import pallas as pl
from jax.experimental.pallas import tpu as pltpu

_T = 0.07
_K_HI = 100.0
_K_LO = 4096.0
_BISECT_ITERS = 27


def _k1_body(ncb, n_cols):
    def body(p_ref, bank_ref, idx_ref, sims_ref, m_ref, pos_ref, accm, accp):
        cb = pl.program_id(1)
        p = p_ref[...]
        npts = p / jnp.sqrt(jnp.sum(p * p, axis=1, keepdims=True))
        bank = bank_ref[...]
        s = jax.lax.dot_general(
            npts, bank, (((1,), (1,)), ((), ())),
            preferred_element_type=jnp.float32,
            precision=jax.lax.Precision.DEFAULT)
        sims_ref[...] = s

        bc = s.shape[1]
        cg = cb * bc + jax.lax.broadcasted_iota(jnp.int32, s.shape, 1)
        valid = cg < n_cols
        sz = jnp.where(valid, s, -jnp.inf)

        @pl.when(cb == 0)
        def _init():
            accm[...] = jnp.full(accm.shape, -jnp.inf, jnp.float32)
            accp[...] = jnp.zeros(accp.shape, jnp.float32)

        accm[...] = jnp.maximum(accm[...], jnp.max(sz, axis=1, keepdims=True))
        hit = (cg == idx_ref[...]) & valid
        accp[...] = accp[...] + jnp.sum(
            jnp.where(hit, jnp.exp(s / _T), 0.0), axis=1, keepdims=True)

        @pl.when(cb == ncb - 1)
        def _fin():
            m_ref[...] = accm[...]
            pos_ref[...] = accp[...]

    return body


def _k2_body(sims_ref, m_ref, pos_ref, out_ref, e_scr):
    s = sims_ref[...]
    m = m_ref[...]
    pos = pos_ref[...]

    lo0 = m - 5.0
    hi0 = m + 0.001

    def bis(_, carry):
        loa, hia, lob, hib = carry
        mida = 0.5 * (loa + hia)
        midb = 0.5 * (lob + hib)
        cnta = jnp.sum((s >= mida).astype(jnp.float32), axis=1,
                       keepdims=True)
        cntb = jnp.sum((s >= midb).astype(jnp.float32), axis=1,
                       keepdims=True)
        preda = cnta >= _K_HI
        predb = cntb >= _K_LO
        return (jnp.where(preda, mida, loa), jnp.where(preda, hia, mida),
                jnp.where(predb, midb, lob), jnp.where(predb, hib, midb))

    loa, hia, lob, hib = jax.lax.fori_loop(
        0, _BISECT_ITERS, bis, (lo0, hi0, lo0, hi0))

    e_scr[...] = jnp.exp(s / _T)
    e = e_scr[...]
    maska = s >= hia
    c_a = jnp.sum(maska.astype(jnp.float32), axis=1, keepdims=True)
    e_a = jnp.sum(jnp.where(maska, e, 0.0), axis=1, keepdims=True)
    s100 = e_a + (_K_HI - c_a) * jnp.exp(loa / _T)
    maskb = s >= hib
    c_b = jnp.sum(maskb.astype(jnp.float32), axis=1, keepdims=True)
    e_b = jnp.sum(jnp.where(maskb, e, 0.0), axis=1, keepdims=True)
    s4096 = e_b + (_K_LO - c_b) * jnp.exp(lob / _T)
    ratio = (pos + s100) / s4096
    term = jnp.log(ratio + 1e-7)
    out_ref[...] = jnp.sum(term).reshape(1, 1, 1)


def _k3_body(n_rows):
    def body(part_ref, out_ref):
        out_ref[...] = (-jnp.sum(part_ref[...]) / n_rows).reshape(1, 1)
    return body


def kernel(points, point_indices, memory_bank):
    b, d = points.shape
    n = memory_bank.shape[0]
    r1 = min(512, b)
    bc = 512
    nrb = b // r1
    ncb = pl.cdiv(n, bc)
    idx2 = point_indices.reshape(b, 1)

    sims, m, pos = pl.pallas_call(
        _k1_body(ncb, n),
        grid=(nrb, ncb),
        in_specs=[
            pl.BlockSpec((r1, d), lambda rb, cb: (rb, 0)),
            pl.BlockSpec((bc, d), lambda rb, cb: (cb, 0)),
            pl.BlockSpec((r1, 1), lambda rb, cb: (rb, 0)),
        ],
        out_specs=[
            pl.BlockSpec((r1, bc), lambda rb, cb: (rb, cb)),
            pl.BlockSpec((r1, 1), lambda rb, cb: (rb, 0)),
            pl.BlockSpec((r1, 1), lambda rb, cb: (rb, 0)),
        ],
        out_shape=[
            jax.ShapeDtypeStruct((b, n), jnp.float32),
            jax.ShapeDtypeStruct((b, 1), jnp.float32),
            jax.ShapeDtypeStruct((b, 1), jnp.float32),
        ],
        scratch_shapes=[
            pltpu.VMEM((r1, 1), jnp.float32),
            pltpu.VMEM((r1, 1), jnp.float32),
        ],
        compiler_params=pltpu.CompilerParams(
            dimension_semantics=("parallel", "arbitrary")),
    )(points, memory_bank, idx2)

    r2 = min(16, b)
    nrb2 = b // r2
    partials = pl.pallas_call(
        _k2_body,
        grid=(nrb2,),
        in_specs=[
            pl.BlockSpec((r2, n), lambda i: (i, 0)),
            pl.BlockSpec((r2, 1), lambda i: (i, 0)),
            pl.BlockSpec((r2, 1), lambda i: (i, 0)),
        ],
        out_specs=pl.BlockSpec((1, 1, 1), lambda i: (i, 0, 0)),
        out_shape=jax.ShapeDtypeStruct((nrb2, 1, 1), jnp.float32),
        scratch_shapes=[pltpu.VMEM((r2, n), jnp.float32)],
        compiler_params=pltpu.CompilerParams(
            dimension_semantics=("parallel",)),
    )(sims, m, pos)

    loss = pl.pallas_call(
        _k3_body(b),
        out_shape=jax.ShapeDtypeStruct((1, 1), jnp.float32),
    )(partials)

    return (loss[0, 0], jax.lax.stop_gradient(sims))


# speedup vs baseline: 22.5346x; 1.5988x over previous
import jax
import jax.numpy as jnp
from jax.experimental import pallas as pl
from jax.experimental.pallas import tpu as pltpu

_T = 0.07
_K_HI = 100.0
_K_LO = 4096.0
_BISECT_JOINT = 10
_BISECT_EXTRA = 5


def _k1_body(ncb, n_cols):
    def body(p_ref, bank_ref, idx_ref, sims_ref, m_ref, pos_ref, accm, accp):
        cb = pl.program_id(1)
        p = p_ref[...]
        npts = p / jnp.sqrt(jnp.sum(p * p, axis=1, keepdims=True))
        bank = bank_ref[...]
        s = jax.lax.dot_general(
            npts, bank, (((1,), (1,)), ((), ())),
            preferred_element_type=jnp.float32,
            precision=jax.lax.Precision.DEFAULT)
        sims_ref[...] = s

        bc = s.shape[1]
        cg = cb * bc + jax.lax.broadcasted_iota(jnp.int32, s.shape, 1)
        valid = cg < n_cols
        sz = jnp.where(valid, s, -jnp.inf)

        @pl.when(cb == 0)
        def _init():
            accm[...] = jnp.full(accm.shape, -jnp.inf, jnp.float32)
            accp[...] = jnp.full(accp.shape, -jnp.inf, jnp.float32)

        accm[...] = jnp.maximum(accm[...], jnp.max(sz, axis=1, keepdims=True))
        hit = (cg == idx_ref[...]) & valid
        accp[...] = jnp.maximum(
            accp[...],
            jnp.max(jnp.where(hit, s, -jnp.inf), axis=1, keepdims=True))

        @pl.when(cb == ncb - 1)
        def _fin():
            m_ref[...] = accm[...]
            pos_ref[...] = jnp.exp(accp[...] / _T)

    return body


def _k2_body(sims_ref, m_ref, pos_ref, out_ref, e_scr):
    s = sims_ref[...]
    m = m_ref[...]
    pos = pos_ref[...]

    lo0 = m - 5.0
    hi0 = m + 0.001

    def bis_joint(_, carry):
        loa, hia, lob, hib = carry
        mida = 0.5 * (loa + hia)
        midb = 0.5 * (lob + hib)
        cnta = jnp.sum((s >= mida).astype(jnp.float32), axis=1,
                       keepdims=True)
        cntb = jnp.sum((s >= midb).astype(jnp.float32), axis=1,
                       keepdims=True)
        preda = cnta >= _K_HI
        predb = cntb >= _K_LO
        return (jnp.where(preda, mida, loa), jnp.where(preda, hia, mida),
                jnp.where(predb, midb, lob), jnp.where(predb, hib, midb))

    def bis_a(_, carry):
        loa, hia = carry
        mida = 0.5 * (loa + hia)
        cnta = jnp.sum((s >= mida).astype(jnp.float32), axis=1,
                       keepdims=True)
        preda = cnta >= _K_HI
        return (jnp.where(preda, mida, loa), jnp.where(preda, hia, mida))

    loa, hia, lob, hib = jax.lax.fori_loop(
        0, _BISECT_JOINT, bis_joint, (lo0, hi0, lo0, hi0))
    loa, hia = jax.lax.fori_loop(0, _BISECT_EXTRA, bis_a, (loa, hia))

    maskb = s >= hib
    e_scr[...] = jnp.where(maskb, jnp.exp(s / _T), 0.0)
    ee = e_scr[...]
    c_b = jnp.sum(maskb.astype(jnp.float32), axis=1, keepdims=True)
    e_b = jnp.sum(ee, axis=1, keepdims=True)
    s4096 = e_b + (_K_LO - c_b) * jnp.exp(lob / _T)
    maska = s >= hia
    c_a = jnp.sum(maska.astype(jnp.float32), axis=1, keepdims=True)
    e_a = jnp.sum(jnp.where(maska, ee, 0.0), axis=1, keepdims=True)
    s100 = e_a + (_K_HI - c_a) * jnp.exp(loa / _T)
    ratio = (pos + s100) / s4096
    term = jnp.log(ratio + 1e-7)
    out_ref[...] = jnp.sum(term).reshape(1, 1, 1)


def _k3_body(n_rows):
    def body(part_ref, out_ref):
        out_ref[...] = (-jnp.sum(part_ref[...]) / n_rows).reshape(1, 1)
    return body


def kernel(points, point_indices, memory_bank):
    b, d = points.shape
    n = memory_bank.shape[0]
    r1 = min(512, b)
    bc = 512
    nrb = b // r1
    ncb = pl.cdiv(n, bc)
    idx2 = point_indices.reshape(b, 1)

    sims, m, pos = pl.pallas_call(
        _k1_body(ncb, n),
        grid=(nrb, ncb),
        in_specs=[
            pl.BlockSpec((r1, d), lambda rb, cb: (rb, 0)),
            pl.BlockSpec((bc, d), lambda rb, cb: (cb, 0)),
            pl.BlockSpec((r1, 1), lambda rb, cb: (rb, 0)),
        ],
        out_specs=[
            pl.BlockSpec((r1, bc), lambda rb, cb: (rb, cb)),
            pl.BlockSpec((r1, 1), lambda rb, cb: (rb, 0)),
            pl.BlockSpec((r1, 1), lambda rb, cb: (rb, 0)),
        ],
        out_shape=[
            jax.ShapeDtypeStruct((b, n), jnp.float32),
            jax.ShapeDtypeStruct((b, 1), jnp.float32),
            jax.ShapeDtypeStruct((b, 1), jnp.float32),
        ],
        scratch_shapes=[
            pltpu.VMEM((r1, 1), jnp.float32),
            pltpu.VMEM((r1, 1), jnp.float32),
        ],
        compiler_params=pltpu.CompilerParams(
            dimension_semantics=("parallel", "arbitrary")),
    )(points, memory_bank, idx2)

    r2 = min(16, b)
    nrb2 = b // r2
    partials = pl.pallas_call(
        _k2_body,
        grid=(nrb2,),
        in_specs=[
            pl.BlockSpec((r2, n), lambda i: (i, 0)),
            pl.BlockSpec((r2, 1), lambda i: (i, 0)),
            pl.BlockSpec((r2, 1), lambda i: (i, 0)),
        ],
        out_specs=pl.BlockSpec((1, 1, 1), lambda i: (i, 0, 0)),
        out_shape=jax.ShapeDtypeStruct((nrb2, 1, 1), jnp.float32),
        scratch_shapes=[pltpu.VMEM((r2, n), jnp.float32)],
        compiler_params=pltpu.CompilerParams(
            dimension_semantics=("parallel",)),
    )(sims, m, pos)

    loss = pl.pallas_call(
        _k3_body(b),
        out_shape=jax.ShapeDtypeStruct((1, 1), jnp.float32),
    )(partials)

    return (loss[0, 0], jax.lax.stop_gradient(sims))


# speedup vs baseline: 22.9497x; 1.0184x over previous
import functools

import jax
import jax.numpy as jnp
from jax import lax
from jax.experimental import pallas as pl
from jax.experimental.pallas import tpu as pltpu
from jax.experimental.pallas import tpu_sc as plsc

_T = 0.07
_K_HI = 100.0
_K_LO = 4096.0
_BISECT_JOINT = 10
_BISECT_EXTRA = 5


def _sc_gather(table, idx):
    info = plsc.get_sparse_core_info()
    nw = info.num_cores * info.num_subcores
    b = idx.shape[0]
    d = table.shape[1]
    b_per_w = b // nw
    mesh = plsc.VectorSubcoreMesh(core_axis_name="c", subcore_axis_name="s")

    @functools.partial(
        pl.kernel, mesh=mesh,
        out_type=jax.ShapeDtypeStruct((b, d), jnp.float32),
        scratch_types=[
            pltpu.VMEM((b_per_w,), jnp.int32),
            pltpu.VMEM((b_per_w, d), jnp.float32),
            pltpu.SemaphoreType.DMA,
        ],
    )
    def gather_kernel(table_hbm, idx_hbm, out_hbm, idx_v, rows_v, sem):
        wid = lax.axis_index("s") * info.num_cores + lax.axis_index("c")
        base = wid * b_per_w
        pltpu.sync_copy(idx_hbm.at[pl.ds(base, b_per_w)], idx_v)
        pltpu.async_copy(table_hbm.at[idx_v], rows_v, sem).wait()
        pltpu.sync_copy(rows_v, out_hbm.at[pl.ds(base, b_per_w)])

    return gather_kernel(table, idx)


def _pos_body(p_ref, g_ref, out_ref):
    p = p_ref[...]
    npts = p / jnp.sqrt(jnp.sum(p * p, axis=1, keepdims=True))
    a = npts.astype(jnp.bfloat16).astype(jnp.float32)
    bb = g_ref[...].astype(jnp.bfloat16).astype(jnp.float32)
    dot = jnp.sum(a * bb, axis=1, keepdims=True)
    out_ref[...] = jnp.exp(dot / _T)


def _k1_body(ncb, n_cols):
    def body(p_ref, bank_ref, sims_ref, m_ref, accm):
        cb = pl.program_id(1)
        p = p_ref[...]
        npts = p / jnp.sqrt(jnp.sum(p * p, axis=1, keepdims=True))
        bank = bank_ref[...]
        s = jax.lax.dot_general(
            npts, bank, (((1,), (1,)), ((), ())),
            preferred_element_type=jnp.float32,
            precision=jax.lax.Precision.DEFAULT)
        sims_ref[...] = s

        bc = s.shape[1]
        cg = cb * bc + jax.lax.broadcasted_iota(jnp.int32, s.shape, 1)
        sz = jnp.where(cg < n_cols, s, -jnp.inf)

        @pl.when(cb == 0)
        def _init():
            accm[...] = jnp.full(accm.shape, -jnp.inf, jnp.float32)

        accm[...] = jnp.maximum(accm[...], jnp.max(sz, axis=1, keepdims=True))

        @pl.when(cb == ncb - 1)
        def _fin():
            m_ref[...] = accm[...]

    return body


def _k2_body(sims_ref, m_ref, pos_ref, out_ref, e_scr):
    s = sims_ref[...]
    m = m_ref[...]
    pos = pos_ref[...]

    lo0 = m - 5.0
    hi0 = m + 0.001

    def bis_joint(_, carry):
        loa, hia, lob, hib = carry
        mida = 0.5 * (loa + hia)
        midb = 0.5 * (lob + hib)
        cnta = jnp.sum((s >= mida).astype(jnp.float32), axis=1,
                       keepdims=True)
        cntb = jnp.sum((s >= midb).astype(jnp.float32), axis=1,
                       keepdims=True)
        preda = cnta >= _K_HI
        predb = cntb >= _K_LO
        return (jnp.where(preda, mida, loa), jnp.where(preda, hia, mida),
                jnp.where(predb, midb, lob), jnp.where(predb, hib, midb))

    def bis_a(_, carry):
        loa, hia = carry
        mida = 0.5 * (loa + hia)
        cnta = jnp.sum((s >= mida).astype(jnp.float32), axis=1,
                       keepdims=True)
        preda = cnta >= _K_HI
        return (jnp.where(preda, mida, loa), jnp.where(preda, hia, mida))

    loa, hia, lob, hib = jax.lax.fori_loop(
        0, _BISECT_JOINT, bis_joint, (lo0, hi0, lo0, hi0))
    loa, hia = jax.lax.fori_loop(0, _BISECT_EXTRA, bis_a, (loa, hia))

    maskb = s >= hib
    e_scr[...] = jnp.where(maskb, jnp.exp(s / _T), 0.0)
    ee = e_scr[...]
    c_b = jnp.sum(maskb.astype(jnp.float32), axis=1, keepdims=True)
    e_b = jnp.sum(ee, axis=1, keepdims=True)
    s4096 = e_b + (_K_LO - c_b) * jnp.exp(lob / _T)
    maska = s >= hia
    c_a = jnp.sum(maska.astype(jnp.float32), axis=1, keepdims=True)
    e_a = jnp.sum(jnp.where(maska, ee, 0.0), axis=1, keepdims=True)
    s100 = e_a + (_K_HI - c_a) * jnp.exp(loa / _T)
    ratio = (pos + s100) / s4096
    term = jnp.log(ratio + 1e-7)
    out_ref[...] = jnp.sum(term).reshape(1, 1, 1)


def _k3_body(n_rows):
    def body(part_ref, out_ref):
        out_ref[...] = (-jnp.sum(part_ref[...]) / n_rows).reshape(1, 1)
    return body


def kernel(points, point_indices, memory_bank):
    b, d = points.shape
    n = memory_bank.shape[0]
    r1 = min(512, b)
    bc = 512
    nrb = b // r1
    ncb = pl.cdiv(n, bc)

    gathered = _sc_gather(memory_bank, point_indices)
    pos = pl.pallas_call(
        _pos_body,
        out_shape=jax.ShapeDtypeStruct((b, 1), jnp.float32),
    )(points, gathered)

    sims, m = pl.pallas_call(
        _k1_body(ncb, n),
        grid=(nrb, ncb),
        in_specs=[
            pl.BlockSpec((r1, d), lambda rb, cb: (rb, 0)),
            pl.BlockSpec((bc, d), lambda rb, cb: (cb, 0)),
        ],
        out_specs=[
            pl.BlockSpec((r1, bc), lambda rb, cb: (rb, cb)),
            pl.BlockSpec((r1, 1), lambda rb, cb: (rb, 0)),
        ],
        out_shape=[
            jax.ShapeDtypeStruct((b, n), jnp.float32),
            jax.ShapeDtypeStruct((b, 1), jnp.float32),
        ],
        scratch_shapes=[
            pltpu.VMEM((r1, 1), jnp.float32),
        ],
        compiler_params=pltpu.CompilerParams(
            dimension_semantics=("parallel", "arbitrary")),
    )(points, memory_bank)

    r2 = min(16, b)
    nrb2 = b // r2
    partials = pl.pallas_call(
        _k2_body,
        grid=(nrb2,),
        in_specs=[
            pl.BlockSpec((r2, n), lambda i: (i, 0)),
            pl.BlockSpec((r2, 1), lambda i: (i, 0)),
            pl.BlockSpec((r2, 1), lambda i: (i, 0)),
        ],
        out_specs=pl.BlockSpec((1, 1, 1), lambda i: (i, 0, 0)),
        out_shape=jax.ShapeDtypeStruct((nrb2, 1, 1), jnp.float32),
        scratch_shapes=[pltpu.VMEM((r2, n), jnp.float32)],
        compiler_params=pltpu.CompilerParams(
            dimension_semantics=("parallel",)),
    )(sims, m, pos)

    loss = pl.pallas_call(
        _k3_body(b),
        out_shape=jax.ShapeDtypeStruct((1, 1), jnp.float32),
    )(partials)

    return (loss[0, 0], jax.lax.stop_gradient(sims))


# speedup vs baseline: 23.6954x; 1.0325x over previous
import functools

import jax
import jax.numpy as jnp
from jax import lax
from jax.experimental import pallas as pl
from jax.experimental.pallas import tpu as pltpu
from jax.experimental.pallas import tpu_sc as plsc

_T = 0.07
_K_HI = 100.0
_K_LO = 4096.0
_BISECT_JOINT = 9
_BISECT_EXTRA = 6


def _sc_gather(table, idx):
    info = plsc.get_sparse_core_info()
    nw = info.num_cores * info.num_subcores
    b = idx.shape[0]
    d = table.shape[1]
    b_per_w = b // nw
    mesh = plsc.VectorSubcoreMesh(core_axis_name="c", subcore_axis_name="s")

    @functools.partial(
        pl.kernel, mesh=mesh,
        out_type=jax.ShapeDtypeStruct((b, d), jnp.float32),
        scratch_types=[
            pltpu.VMEM((b_per_w,), jnp.int32),
            pltpu.VMEM((b_per_w, d), jnp.float32),
            pltpu.SemaphoreType.DMA,
        ],
    )
    def gather_kernel(table_hbm, idx_hbm, out_hbm, idx_v, rows_v, sem):
        wid = lax.axis_index("s") * info.num_cores + lax.axis_index("c")
        base = wid * b_per_w
        pltpu.sync_copy(idx_hbm.at[pl.ds(base, b_per_w)], idx_v)
        pltpu.async_copy(table_hbm.at[idx_v], rows_v, sem).wait()
        pltpu.sync_copy(rows_v, out_hbm.at[pl.ds(base, b_per_w)])

    return gather_kernel(table, idx)


def _pos_body(p_ref, g_ref, out_ref):
    p = p_ref[...]
    npts = p / jnp.sqrt(jnp.sum(p * p, axis=1, keepdims=True))
    a = npts.astype(jnp.bfloat16).astype(jnp.float32)
    bb = g_ref[...].astype(jnp.bfloat16).astype(jnp.float32)
    dot = jnp.sum(a * bb, axis=1, keepdims=True)
    out_ref[...] = jnp.exp(dot / _T)


def _k1_body(ncb, n_cols):
    def body(p_ref, bank_ref, sims_ref, m_ref, accm):
        cb = pl.program_id(1)
        p = p_ref[...]
        npts = p / jnp.sqrt(jnp.sum(p * p, axis=1, keepdims=True))
        bank = bank_ref[...]
        s = jax.lax.dot_general(
            npts, bank, (((1,), (1,)), ((), ())),
            preferred_element_type=jnp.float32,
            precision=jax.lax.Precision.DEFAULT)
        sims_ref[...] = s

        bc = s.shape[1]
        cg = cb * bc + jax.lax.broadcasted_iota(jnp.int32, s.shape, 1)
        sz = jnp.where(cg < n_cols, s, -jnp.inf)

        @pl.when(cb == 0)
        def _init():
            accm[...] = jnp.full(accm.shape, -jnp.inf, jnp.float32)

        accm[...] = jnp.maximum(accm[...], jnp.max(sz, axis=1, keepdims=True))

        @pl.when(cb == ncb - 1)
        def _fin():
            m_ref[...] = accm[...]

    return body


def _k2_body(sims_ref, m_ref, pos_ref, out_ref, e_scr):
    s = sims_ref[...]
    m = m_ref[...]
    pos = pos_ref[...]

    lo0 = m - 5.0
    hi0 = m + 0.001

    def bis_joint(_, carry):
        loa, hia, lob, hib = carry
        mida = 0.5 * (loa + hia)
        midb = 0.5 * (lob + hib)
        cnta = jnp.sum((s >= mida).astype(jnp.float32), axis=1,
                       keepdims=True)
        cntb = jnp.sum((s >= midb).astype(jnp.float32), axis=1,
                       keepdims=True)
        preda = cnta >= _K_HI
        predb = cntb >= _K_LO
        return (jnp.where(preda, mida, loa), jnp.where(preda, hia, mida),
                jnp.where(predb, midb, lob), jnp.where(predb, hib, midb))

    def bis_a(_, carry):
        loa, hia = carry
        mida = 0.5 * (loa + hia)
        cnta = jnp.sum((s >= mida).astype(jnp.float32), axis=1,
                       keepdims=True)
        preda = cnta >= _K_HI
        return (jnp.where(preda, mida, loa), jnp.where(preda, hia, mida))

    loa, hia, lob, hib = jax.lax.fori_loop(
        0, _BISECT_JOINT, bis_joint, (lo0, hi0, lo0, hi0))
    loa, hia = jax.lax.fori_loop(0, _BISECT_EXTRA, bis_a, (loa, hia))

    e_scr[...] = jnp.where(s >= hib, jnp.exp(s / _T), 0.0)
    ee = e_scr[...]
    s4096 = jnp.sum(ee, axis=1, keepdims=True)
    maska = s >= hia
    c_a = jnp.sum(maska.astype(jnp.float32), axis=1, keepdims=True)
    e_a = jnp.sum(jnp.where(maska, ee, 0.0), axis=1, keepdims=True)
    s100 = e_a + (_K_HI - c_a) * jnp.exp(loa / _T)
    ratio = (pos + s100) / s4096
    term = jnp.log(ratio + 1e-7)
    out_ref[...] = jnp.sum(term).reshape(1, 1, 1)


def _k3_body(n_rows):
    def body(part_ref, out_ref):
        out_ref[...] = (-jnp.sum(part_ref[...]) / n_rows).reshape(1, 1)
    return body


def kernel(points, point_indices, memory_bank):
    b, d = points.shape
    n = memory_bank.shape[0]
    r1 = min(512, b)
    bc = 512
    nrb = b // r1
    ncb = pl.cdiv(n, bc)

    gathered = _sc_gather(memory_bank, point_indices)
    pos = pl.pallas_call(
        _pos_body,
        out_shape=jax.ShapeDtypeStruct((b, 1), jnp.float32),
    )(points, gathered)

    sims, m = pl.pallas_call(
        _k1_body(ncb, n),
        grid=(nrb, ncb),
        in_specs=[
            pl.BlockSpec((r1, d), lambda rb, cb: (rb, 0)),
            pl.BlockSpec((bc, d), lambda rb, cb: (cb, 0)),
        ],
        out_specs=[
            pl.BlockSpec((r1, bc), lambda rb, cb: (rb, cb)),
            pl.BlockSpec((r1, 1), lambda rb, cb: (rb, 0)),
        ],
        out_shape=[
            jax.ShapeDtypeStruct((b, n), jnp.float32),
            jax.ShapeDtypeStruct((b, 1), jnp.float32),
        ],
        scratch_shapes=[
            pltpu.VMEM((r1, 1), jnp.float32),
        ],
        compiler_params=pltpu.CompilerParams(
            dimension_semantics=("parallel", "arbitrary")),
    )(points, memory_bank)

    r2 = min(16, b)
    nrb2 = b // r2
    partials = pl.pallas_call(
        _k2_body,
        grid=(nrb2,),
        in_specs=[
            pl.BlockSpec((r2, n), lambda i: (i, 0)),
            pl.BlockSpec((r2, 1), lambda i: (i, 0)),
            pl.BlockSpec((r2, 1), lambda i: (i, 0)),
        ],
        out_specs=pl.BlockSpec((1, 1, 1), lambda i: (i, 0, 0)),
        out_shape=jax.ShapeDtypeStruct((nrb2, 1, 1), jnp.float32),
        scratch_shapes=[pltpu.VMEM((r2, n), jnp.float32)],
        compiler_params=pltpu.CompilerParams(
            dimension_semantics=("parallel",)),
    )(sims, m, pos)

    loss = pl.pallas_call(
        _k3_body(b),
        out_shape=jax.ShapeDtypeStruct((1, 1), jnp.float32),
    )(partials)

    return (loss[0, 0], jax.lax.stop_gradient(sims))


# speedup vs baseline: 26.4801x; 1.1175x over previous
import functools

import jax
import jax.numpy as jnp
from jax import lax
from jax.experimental import pallas as pl
from jax.experimental.pallas import tpu as pltpu
from jax.experimental.pallas import tpu_sc as plsc

_T = 0.07
_K_HI = 100.0
_K_LO = 4096.0
_BISECT_JOINT = 9
_BISECT_EXTRA = 6


def _sc_gather(table, idx):
    info = plsc.get_sparse_core_info()
    nw = info.num_cores * info.num_subcores
    b = idx.shape[0]
    d = table.shape[1]
    b_per_w = b // nw
    mesh = plsc.VectorSubcoreMesh(core_axis_name="c", subcore_axis_name="s")

    @functools.partial(
        pl.kernel, mesh=mesh,
        out_type=jax.ShapeDtypeStruct((b, d), jnp.float32),
        scratch_types=[
            pltpu.VMEM((b_per_w,), jnp.int32),
            pltpu.VMEM((b_per_w, d), jnp.float32),
            pltpu.SemaphoreType.DMA,
        ],
    )
    def gather_kernel(table_hbm, idx_hbm, out_hbm, idx_v, rows_v, sem):
        wid = lax.axis_index("s") * info.num_cores + lax.axis_index("c")
        base = wid * b_per_w
        pltpu.sync_copy(idx_hbm.at[pl.ds(base, b_per_w)], idx_v)
        pltpu.async_copy(table_hbm.at[idx_v], rows_v, sem).wait()
        pltpu.sync_copy(rows_v, out_hbm.at[pl.ds(base, b_per_w)])

    return gather_kernel(table, idx)


def _pos_body(p_ref, g_ref, out_ref):
    p = p_ref[...]
    npts = p / jnp.sqrt(jnp.sum(p * p, axis=1, keepdims=True))
    a = npts.astype(jnp.bfloat16).astype(jnp.float32)
    bb = g_ref[...].astype(jnp.bfloat16).astype(jnp.float32)
    dot = jnp.sum(a * bb, axis=1, keepdims=True)
    out_ref[...] = jnp.exp(dot / _T)


def _k1_body(ncb, n_cols):
    def body(p_ref, bank_ref, sims_ref, m_ref, accm):
        cb = pl.program_id(1)
        p = p_ref[...]
        npts = p / jnp.sqrt(jnp.sum(p * p, axis=1, keepdims=True))
        bank = bank_ref[...]
        s = jax.lax.dot_general(
            npts, bank, (((1,), (1,)), ((), ())),
            preferred_element_type=jnp.float32,
            precision=jax.lax.Precision.DEFAULT)
        sims_ref[...] = s

        bc = s.shape[1]
        cg = cb * bc + jax.lax.broadcasted_iota(jnp.int32, s.shape, 1)
        sz = jnp.where(cg < n_cols, s, -jnp.inf)

        @pl.when(cb == 0)
        def _init():
            accm[...] = jnp.full(accm.shape, -jnp.inf, jnp.float32)

        accm[...] = jnp.maximum(accm[...], jnp.max(sz, axis=1, keepdims=True))

        @pl.when(cb == ncb - 1)
        def _fin():
            m_ref[...] = accm[...]

    return body


def _k2_body(sims_ref, m_ref, pos_ref, out_ref, e_scr):
    s = sims_ref[...]
    m = m_ref[...]
    pos = pos_ref[...]

    lo0 = m - 5.0
    hi0 = m + 0.001

    def bis_joint(_, carry):
        loa, hia, lob, hib = carry
        mida = 0.5 * (loa + hia)
        midb = 0.5 * (lob + hib)
        cnta = jnp.sum((s >= mida).astype(jnp.float32), axis=1,
                       keepdims=True)
        cntb = jnp.sum((s >= midb).astype(jnp.float32), axis=1,
                       keepdims=True)
        preda = cnta >= _K_HI
        predb = cntb >= _K_LO
        return (jnp.where(preda, mida, loa), jnp.where(preda, hia, mida),
                jnp.where(predb, midb, lob), jnp.where(predb, hib, midb))

    def bis_a(_, carry):
        loa, hia = carry
        mida = 0.5 * (loa + hia)
        cnta = jnp.sum((s >= mida).astype(jnp.float32), axis=1,
                       keepdims=True)
        preda = cnta >= _K_HI
        return (jnp.where(preda, mida, loa), jnp.where(preda, hia, mida))

    loa, hia, lob, hib = jax.lax.fori_loop(
        0, _BISECT_JOINT, bis_joint, (lo0, hi0, lo0, hi0))
    loa, hia = jax.lax.fori_loop(0, _BISECT_EXTRA, bis_a, (loa, hia))

    e_scr[...] = jnp.where(s >= hib, jnp.exp(s / _T), 0.0)
    ee = e_scr[...]
    s4096 = jnp.sum(ee, axis=1, keepdims=True)
    maska = s >= hia
    c_a = jnp.sum(maska.astype(jnp.float32), axis=1, keepdims=True)
    e_a = jnp.sum(jnp.where(maska, ee, 0.0), axis=1, keepdims=True)
    s100 = e_a + (_K_HI - c_a) * jnp.exp(loa / _T)
    ratio = (pos + s100) / s4096
    term = jnp.log(ratio + 1e-7)
    out_ref[...] = jnp.sum(term).reshape(1, 1, 1)


def _k3_body(n_rows):
    def body(part_ref, out_ref):
        out_ref[...] = (-jnp.sum(part_ref[...]) / n_rows).reshape(1, 1)
    return body


def kernel(points, point_indices, memory_bank):
    b, d = points.shape
    n = memory_bank.shape[0]
    r1 = min(512, b)
    bc = 512
    nrb = b // r1
    ncb = pl.cdiv(n, bc)

    gathered = _sc_gather(memory_bank, point_indices)
    pos = pl.pallas_call(
        _pos_body,
        out_shape=jax.ShapeDtypeStruct((b, 1), jnp.float32),
    )(points, gathered)

    sims, m = pl.pallas_call(
        _k1_body(ncb, n),
        grid=(nrb, ncb),
        in_specs=[
            pl.BlockSpec((r1, d), lambda rb, cb: (rb, 0)),
            pl.BlockSpec((bc, d), lambda rb, cb: (cb, 0)),
        ],
        out_specs=[
            pl.BlockSpec((r1, bc), lambda rb, cb: (rb, cb)),
            pl.BlockSpec((r1, 1), lambda rb, cb: (rb, 0)),
        ],
        out_shape=[
            jax.ShapeDtypeStruct((b, n), jnp.float32),
            jax.ShapeDtypeStruct((b, 1), jnp.float32),
        ],
        scratch_shapes=[
            pltpu.VMEM((r1, 1), jnp.float32),
        ],
        compiler_params=pltpu.CompilerParams(
            dimension_semantics=("parallel", "arbitrary")),
    )(points, memory_bank)

    r2 = min(32, b)
    nrb2 = b // r2
    partials = pl.pallas_call(
        _k2_body,
        grid=(nrb2,),
        in_specs=[
            pl.BlockSpec((r2, n), lambda i: (i, 0)),
            pl.BlockSpec((r2, 1), lambda i: (i, 0)),
            pl.BlockSpec((r2, 1), lambda i: (i, 0)),
        ],
        out_specs=pl.BlockSpec((1, 1, 1), lambda i: (i, 0, 0)),
        out_shape=jax.ShapeDtypeStruct((nrb2, 1, 1), jnp.float32),
        scratch_shapes=[pltpu.VMEM((r2, n), jnp.float32)],
        compiler_params=pltpu.CompilerParams(
            dimension_semantics=("parallel",)),
    )(sims, m, pos)

    loss = pl.pallas_call(
        _k3_body(b),
        out_shape=jax.ShapeDtypeStruct((1, 1), jnp.float32),
    )(partials)

    return (loss[0, 0], jax.lax.stop_gradient(sims))
